# baseline (speedup 1.0000x reference)
import functools
import jax
import jax.numpy as jnp
import numpy as np
from jax import lax
from jax.experimental import pallas as pl
from jax.experimental.pallas import tpu as pltpu
from jax.experimental.pallas import tpu_sc as plsc

F = 128
NC, NS, LANES = 2, 16, 16
NW = NC * NS
C = 48
NG = 256
F2 = F // 2
TP = 3072
HID = 64

_PREC = jax.lax.Precision.HIGHEST


def _dot(a, b):
    return jax.lax.dot_general(a, b, (((a.ndim - 1,), (0,)), ((), ())),
                               preferred_element_type=jnp.float32,
                               precision=_PREC)


def _sp(x, beta, thr):
    z = jnp.minimum(x * beta, thr)
    return jnp.where(x * beta > thr, x, (1.0 / beta) * jnp.log(1.0 + jnp.exp(z)))


def _pack64(x):
    ul = pltpu.bitcast(x[:, :F2], jnp.uint32)
    uh = pltpu.bitcast(x[:, F2:], jnp.uint32)
    rl = (ul + 0x7FFF + ((ul >> 16) & 1)) >> 16
    rh = (uh + 0x7FFF + ((uh >> 16) & 1)) >> 16
    return pltpu.bitcast(rl | (rh << 16), jnp.float32)


def _tables_body(ee, c3W, c3b, e1W, e1b, o0, o1, o2):
    T = ee[...]
    outs = [o0, o1, o2]
    for l in range(3):
        Tn = _dot(T, c3W[l]) + c3b[l]
        outs[l][...] = Tn
        T = _sp(_dot(Tn, e1W[l]) + e1b[l], 0.5, 14.0)


def _rh_body(inv_gap, d, cent, c1W, c1b, c2W, c2b, o0, o1, o2):
    dv = d[...]
    rbf = jnp.exp(-((dv - cent[...]) ** 2) * inv_gap)
    outs = [o0, o1, o2]
    for l in range(3):
        r1 = _sp(_dot(rbf, c1W[l]) + c1b[l], 0.5, 14.0)
        outs[l][...] = _pack64(_dot(r1, c2W[l]) + c2b[l])


def _h0_body(nt, emb, out):
    B = nt.shape[0]
    oh = (nt[...] == lax.broadcasted_iota(jnp.int32, (B, F), 1)).astype(jnp.float32)
    out[...] = _dot(oh, emb[...])


def _pre_body(h, W, b, out):
    out[...] = _dot(h[...], W[...]) + b[...]


def _post_body(agg2, h, W2, b2, W3, b3, hn):
    a = agg2[0] + agg2[1]
    nm = _sp(_dot(a, W2[...]) + b2[...], 0.5, 14.0)
    hn[...] = h[...] + _dot(nm, W3[...]) + b3[...]


def _readout_body(h0, h1, h2, h3, gid, W4, b, W2, out, acc):
    i = pl.program_id(0)
    n = pl.num_programs(0)
    B = h0.shape[0]
    hr = _dot(h0[...], W4[0]) + _dot(h1[...], W4[1]) + \
         _dot(h2[...], W4[2]) + _dot(h3[...], W4[3]) + b[...]
    hr = _sp(hr, 1.0, 20.0)
    hrp = jnp.concatenate([hr, jnp.ones((B, HID), jnp.float32)], axis=1)
    seg = (lax.broadcasted_iota(jnp.int32, (NG, B), 0) == gid[0]).astype(jnp.float32)

    @pl.when(i == 0)
    def _():
        acc[...] = jnp.zeros_like(acc)

    acc[...] += _dot(seg, hrp)

    @pl.when(i == n - 1)
    def _():
        out[...] = _dot(acc[...], W2[...])


def _etype_sc(VP, EP, K):
    mesh = plsc.VectorSubcoreMesh(core_axis_name="c", subcore_axis_name="s",
                                  num_cores=NC, num_subcores=NS)

    @functools.partial(
        pl.kernel, mesh=mesh,
        compiler_params=pltpu.CompilerParams(needs_layout_passes=False),
        out_type=jax.ShapeDtypeStruct((EP,), jnp.int32),
        scratch_types=[
            pltpu.VMEM((VP,), jnp.int32),
            pltpu.VMEM((C,), jnp.int32),
            pltpu.VMEM((C,), jnp.int32),
            pltpu.VMEM((C,), jnp.int32),
        ],
    )
    def k(nt_hbm, src_hbm, dst_hbm, et_out, nt_v, s_v, d_v, e_v):
        cid = lax.axis_index("c")
        sid = lax.axis_index("s")
        wid = sid * NC + cid
        pltpu.sync_copy(nt_hbm, nt_v)

        def chunk(kk, _):
            base = wid * (K * C) + kk * C
            pltpu.sync_copy(src_hbm.at[pl.ds(base, C)], s_v)
            pltpu.sync_copy(dst_hbm.at[pl.ds(base, C)], d_v)

            def grp(j, _):
                sl = pl.ds(j * LANES, LANES)
                tx = plsc.load_gather(nt_v, [s_v[sl]])
                ty = plsc.load_gather(nt_v, [d_v[sl]])
                e_v[sl] = tx * ty + tx + ty
                return 0

            lax.fori_loop(0, C // LANES, grp, 0)
            pltpu.sync_copy(e_v, et_out.at[pl.ds(base, C)])
            return 0

        lax.fori_loop(0, K, chunk, 0)

    return k


def _edge_sc(VPS, EP, K):
    mesh = plsc.VectorSubcoreMesh(core_axis_name="c", subcore_axis_name="s",
                                  num_cores=NC, num_subcores=NS)
    rows = VPS // NS
    KQ = K // 4

    @functools.partial(
        pl.kernel, mesh=mesh,
        compiler_params=pltpu.CompilerParams(needs_layout_passes=False),
        out_type=jax.ShapeDtypeStruct((NC, VPS, F), jnp.float32),
        scratch_types=[
            pltpu.VMEM((4, C), jnp.int32),
            pltpu.VMEM((4, C), jnp.int32),
            pltpu.VMEM((4, C), jnp.int32),
            pltpu.VMEM((2, C, F), jnp.float32),
            pltpu.VMEM((2, C, F), jnp.float32),
            pltpu.VMEM((2, C, F2), jnp.float32),
            pltpu.VMEM_SHARED((VPS, F), jnp.float32),
        ] + [pltpu.SemaphoreType.DMA] * 10,
    )
    def k(nn_hbm, rh_hbm, tnew_hbm, src_hbm, dst_hbm, et_hbm, zeros_hbm, agg_out,
          s4, d4, e4, hs2, tf2, rr2, shared,
          si0, si1, si2, si3, sr0, sr1, sg0, sg1, ss0, ss1):
        cid = lax.axis_index("c")
        sid = lax.axis_index("s")
        wid = sid * NC + cid
        si = [si0, si1, si2, si3]
        sr = [sr0, sr1]
        sg = [sg0, sg1]
        ss = [ss0, ss1]
        ebase = wid * (K * C)

        def issue_idx(kq, j, sem):
            base = ebase + kq * C
            pltpu.async_copy(src_hbm.at[pl.ds(base, C)], s4.at[j], sem)
            pltpu.async_copy(dst_hbm.at[pl.ds(base, C)], d4.at[j], sem)
            pltpu.async_copy(et_hbm.at[pl.ds(base, C)], e4.at[j], sem)

        def wait_idx(kq, j, sem):
            base = ebase + kq * C
            pltpu.make_async_copy(src_hbm.at[pl.ds(base, C)], s4.at[j], sem).wait()
            pltpu.make_async_copy(dst_hbm.at[pl.ds(base, C)], d4.at[j], sem).wait()
            pltpu.make_async_copy(et_hbm.at[pl.ds(base, C)], e4.at[j], sem).wait()

        def issue_rh(kq, b, sem):
            base = ebase + kq * C
            pltpu.async_copy(rh_hbm.at[pl.ds(base, C)], rr2.at[b], sem)

        def wait_rh(kq, b, sem):
            base = ebase + kq * C
            pltpu.make_async_copy(rh_hbm.at[pl.ds(base, C)], rr2.at[b], sem).wait()

        def wait_scatter(j, b, sem):
            pltpu.make_async_copy(tf2.at[b], shared.at[d4.at[j]], sem).wait()

        pltpu.sync_copy(zeros_hbm.at[pl.ds(sid * rows, rows)],
                        shared.at[pl.ds(sid * rows, rows)])
        plsc.subcore_barrier()

        issue_idx(0, 0, si[0])
        issue_idx(1, 1, si[1])
        issue_rh(0, 0, sr[0])
        issue_rh(1, 1, sr[1])

        def outer(kk, _):
            for b in range(4):
                kq = kk * 4 + b
                b2 = b % 2

                def _wait_sc():
                    wait_scatter((b + 2) % 4, b2, ss[b2])
                if b < 2:
                    pl.when(kk > 0)(_wait_sc)
                else:
                    _wait_sc()

                def _pf_idx():
                    issue_idx(kq + 2, (b + 2) % 4, si[(b + 2) % 4])
                if b < 2:
                    _pf_idx()
                else:
                    pl.when(kk < KQ - 1)(_pf_idx)

                wait_idx(kq, b, si[b])

                g1 = pltpu.async_copy(nn_hbm.at[s4.at[b]], hs2.at[b2], sg[b2])
                g2 = pltpu.async_copy(tnew_hbm.at[e4.at[b]], tf2.at[b2], sg[b2])

                wait_rh(kq, b2, sr[b2])

                g1.wait()
                g2.wait()

                @plsc.parallel_loop(0, C, unroll=2)
                def _row(r):
                    for g in range(F2 // LANES):
                        rlo, rhi = plsc.unpack(
                            plsc.bitcast(rr2[b2, r, pl.ds(g * LANES, LANES)],
                                         jnp.bfloat16),
                            format=plsc.PackFormat.INTERLEAVED,
                            preferred_element_type=jnp.float32)
                        slo = pl.ds(g * LANES, LANES)
                        shi = pl.ds(F2 + g * LANES, LANES)
                        tf2[b2, r, slo] = hs2[b2, r, slo] * rlo + tf2[b2, r, slo]
                        tf2[b2, r, shi] = hs2[b2, r, shi] * rhi + tf2[b2, r, shi]

                pltpu.async_copy(tf2.at[b2], shared.at[d4.at[b]], ss[b2],
                                 add=True)

                def _pf_rh():
                    issue_rh(kq + 2, b2, sr[b2])
                if b < 2:
                    _pf_rh()
                else:
                    pl.when(kk < KQ - 1)(_pf_rh)
            return 0

        lax.fori_loop(0, KQ, outer, 0)
        wait_scatter(2, 0, ss[0])
        wait_scatter(3, 1, ss[1])
        plsc.subcore_barrier()
        pltpu.sync_copy(shared.at[pl.ds(sid * rows, rows)],
                        agg_out.at[cid, pl.ds(sid * rows, rows)])

    return k


def _pad_rows(x, n):
    return jnp.concatenate(
        [x, jnp.zeros((n - x.shape[0],) + x.shape[1:], x.dtype)], axis=0)


@jax.jit
def _run(node_types, edge_index, edge_dists, node_graph_ids, params):
    V = node_types.shape[0]
    E = edge_index.shape[1]
    VP = ((V + 1 + 1023) // 1024) * 1024
    VPS = ((V + 1 + 127) // 128) * 128
    K = ((-(-E // (NW * C)) + 3) // 4) * 4
    EP = NW * C * K

    nt = node_types.astype(jnp.int32)
    src = _pad_rows(edge_index[0].astype(jnp.int32), EP)
    dst = jnp.concatenate([edge_index[1].astype(jnp.int32),
                           jnp.full((EP - E,), VPS - 1, jnp.int32)])
    dists = _pad_rows(edge_dists[:, 0], EP).reshape(EP, 1)
    ntp = _pad_rows(nt, VP)
    gid = jnp.concatenate([node_graph_ids.astype(jnp.int32),
                           jnp.full((VP - V,), NG + 7, jnp.int32)])

    lps = params['layers']
    c3W = jnp.stack([lp['c3_W'] for lp in lps])
    c3b = jnp.stack([lp['c3_b'] for lp in lps])
    e1W = jnp.stack([lp['el1_W'] for lp in lps])
    e1b = jnp.stack([lp['el1_b'] for lp in lps])
    c1W = jnp.stack([_pad_rows(lp['c1_W'], 8) for lp in lps])
    c1b = jnp.stack([lp['c1_b'] for lp in lps])
    c2W = jnp.stack([lp['c2_W'] for lp in lps])
    c2b = jnp.stack([lp['c2_b'] for lp in lps])
    ee = _pad_rows(params['edge_embed'], TP)
    emb = _pad_rows(params['node_embed'], F)

    RBF_DIM = 5
    centers = np.linspace(0.0, 5.0, RBF_DIM).astype(np.float32)
    inv_gap = float(1.0 / (centers[1] - centers[0]))
    cent = jnp.asarray(np.pad(centers, (0, 8 - RBF_DIM)))

    f32 = jnp.float32
    tnews = pl.pallas_call(
        _tables_body,
        out_shape=[jax.ShapeDtypeStruct((TP, F), f32)] * 3,
    )(ee, c3W, c3b, e1W, e1b)

    BR = 2048
    nblk = EP // BR
    rhs = pl.pallas_call(
        functools.partial(_rh_body, inv_gap),
        grid=(nblk,),
        in_specs=[
            pl.BlockSpec((BR, 1), lambda i: (i, 0)),
            pl.BlockSpec((8,), lambda i: (0,)),
            pl.BlockSpec((3, 8, F), lambda i: (0, 0, 0)),
            pl.BlockSpec((3, F), lambda i: (0, 0)),
            pl.BlockSpec((3, F, F), lambda i: (0, 0, 0)),
            pl.BlockSpec((3, F), lambda i: (0, 0)),
        ],
        out_specs=[pl.BlockSpec((BR, F2), lambda i: (i, 0))] * 3,
        out_shape=[jax.ShapeDtypeStruct((EP, F2), f32)] * 3,
    )(dists, cent, c1W, c1b, c2W, c2b)

    etype = _etype_sc(VP, EP, K)(ntp, src, dst)

    BN = 2048
    nbn = VP // BN
    h0 = pl.pallas_call(
        _h0_body,
        grid=(nbn,),
        in_specs=[
            pl.BlockSpec((BN, 1), lambda i: (i, 0)),
            pl.BlockSpec((F, F), lambda i: (0, 0)),
        ],
        out_specs=pl.BlockSpec((BN, F), lambda i: (i, 0)),
        out_shape=jax.ShapeDtypeStruct((VP, F), f32),
    )(ntp.reshape(VP, 1), emb)

    zeros_vp = jnp.zeros((VPS, F), f32)
    edge_k = _edge_sc(VPS, EP, K)

    h = h0
    feats = [h0]
    for l in range(3):
        lp = lps[l]
        nn = pl.pallas_call(
            _pre_body,
            grid=(nbn,),
            in_specs=[
                pl.BlockSpec((BN, F), lambda i: (i, 0)),
                pl.BlockSpec((F, F), lambda i: (0, 0)),
                pl.BlockSpec((F,), lambda i: (0,)),
            ],
            out_specs=pl.BlockSpec((BN, F), lambda i: (i, 0)),
            out_shape=jax.ShapeDtypeStruct((VP, F), f32),
        )(h, lp['nl1_W'], lp['nl1_b'])

        agg2 = edge_k(nn, rhs[l], tnews[l], src, dst, etype, zeros_vp)
        agg2 = jnp.concatenate(
            [agg2, jnp.zeros((NC, VP - VPS, F), f32)], axis=1)

        h = pl.pallas_call(
            _post_body,
            grid=(nbn,),
            in_specs=[
                pl.BlockSpec((NC, BN, F), lambda i: (0, i, 0)),
                pl.BlockSpec((BN, F), lambda i: (i, 0)),
                pl.BlockSpec((F, F), lambda i: (0, 0)),
                pl.BlockSpec((F,), lambda i: (0,)),
                pl.BlockSpec((F, F), lambda i: (0, 0)),
                pl.BlockSpec((F,), lambda i: (0,)),
            ],
            out_specs=pl.BlockSpec((BN, F), lambda i: (i, 0)),
            out_shape=jax.ShapeDtypeStruct((VP, F), f32),
        )(agg2, h, lp['nl2_W'], lp['nl2_b'], lp['nl3_W'], lp['nl3_b'])
        feats.append(h)

    W4 = params['ro_in_W'].reshape(4, F, HID)
    W2 = jnp.zeros((F, F), f32)
    W2 = W2.at[:HID, 0].set(params['ro_out_W'][:, 0])
    W2 = W2.at[HID, 0].set(params['ro_out_b'][0])
    BG = 1024
    nbg = VP // BG
    gid3 = gid.reshape(nbg, 1, BG)
    out = pl.pallas_call(
        _readout_body,
        grid=(nbg,),
        in_specs=[pl.BlockSpec((BG, F), lambda i: (i, 0))] * 4 + [
            pl.BlockSpec((1, 1, BG), lambda i: (i, 0, 0)),
            pl.BlockSpec((4, F, HID), lambda i: (0, 0, 0)),
            pl.BlockSpec((HID,), lambda i: (0,)),
            pl.BlockSpec((F, F), lambda i: (0, 0)),
        ],
        out_specs=pl.BlockSpec((NG, F), lambda i: (0, 0)),
        out_shape=jax.ShapeDtypeStruct((NG, F), f32),
        scratch_shapes=[pltpu.VMEM((NG, F), f32)],
    )(feats[0], feats[1], feats[2], feats[3], gid3, W4,
      params['ro_in_b'], W2)
    return out[:, :1]


def kernel(node_types, edge_index, edge_dists, node_graph_ids, params):
    return _run(node_types, edge_index, edge_dists, node_graph_ids, params)


# speedup vs baseline: 1.4054x; 1.4054x over previous
import functools
import jax
import jax.numpy as jnp
import numpy as np
from jax import lax
from jax.experimental import pallas as pl
from jax.experimental.pallas import tpu as pltpu
from jax.experimental.pallas import tpu_sc as plsc

F = 128
NC, NS, LANES = 2, 16, 16
NW = NC * NS
C = 32
NG = 256
F2 = F // 2
TP = 3072
HID = 64

_PREC = jax.lax.Precision.HIGHEST


def _dot(a, b):
    return jax.lax.dot_general(a, b, (((a.ndim - 1,), (0,)), ((), ())),
                               preferred_element_type=jnp.float32,
                               precision=_PREC)


def _sp(x, beta, thr):
    z = jnp.minimum(x * beta, thr)
    return jnp.where(x * beta > thr, x, (1.0 / beta) * jnp.log(1.0 + jnp.exp(z)))


def _tables_body(ee, c3W, c3b, e1W, e1b, o0, o1, o2):
    T = ee[...]
    outs = [o0, o1, o2]
    for l in range(3):
        Tn = _dot(T, c3W[l]) + c3b[l]
        outs[l][...] = Tn
        T = _sp(_dot(Tn, e1W[l]) + e1b[l], 0.5, 14.0)


def _rh_body(inv_gap, d, cent, c1W, c1b, c2W, c2b, o0, o1, o2):
    dv = d[...]
    rbf = jnp.exp(-((dv - cent[...]) ** 2) * inv_gap)
    outs = [o0, o1, o2]
    for l in range(3):
        r1 = _sp(_dot(rbf, c1W[l]) + c1b[l], 0.5, 14.0)
        outs[l][...] = _dot(r1, c2W[l]) + c2b[l]


def _h0_body(nt, emb, out):
    B = nt.shape[0]
    oh = (nt[...] == lax.broadcasted_iota(jnp.int32, (B, F), 1)).astype(jnp.float32)
    out[...] = _dot(oh, emb[...])


def _pre_body(h, W, b, out):
    out[...] = _dot(h[...], W[...]) + b[...]


def _post_body(agg2, h, W2, b2, W3, b3, hn):
    a = agg2[0] + agg2[1]
    nm = _sp(_dot(a, W2[...]) + b2[...], 0.5, 14.0)
    hn[...] = h[...] + _dot(nm, W3[...]) + b3[...]


def _readout_body(h0, h1, h2, h3, gid, W4, b, W2, out, acc):
    i = pl.program_id(0)
    n = pl.num_programs(0)
    B = h0.shape[0]
    hr = _dot(h0[...], W4[0]) + _dot(h1[...], W4[1]) + \
         _dot(h2[...], W4[2]) + _dot(h3[...], W4[3]) + b[...]
    hr = _sp(hr, 1.0, 20.0)
    hrp = jnp.concatenate([hr, jnp.ones((B, HID), jnp.float32)], axis=1)
    seg = (lax.broadcasted_iota(jnp.int32, (NG, B), 0) == gid[0]).astype(jnp.float32)

    @pl.when(i == 0)
    def _():
        acc[...] = jnp.zeros_like(acc)

    acc[...] += _dot(seg, hrp)

    @pl.when(i == n - 1)
    def _():
        out[...] = _dot(acc[...], W2[...])


def _etype_sc(VP, EP, K):
    mesh = plsc.VectorSubcoreMesh(core_axis_name="c", subcore_axis_name="s",
                                  num_cores=NC, num_subcores=NS)

    @functools.partial(
        pl.kernel, mesh=mesh,
        compiler_params=pltpu.CompilerParams(needs_layout_passes=False),
        out_type=jax.ShapeDtypeStruct((EP,), jnp.int32),
        scratch_types=[
            pltpu.VMEM((VP,), jnp.int32),
            pltpu.VMEM((C,), jnp.int32),
            pltpu.VMEM((C,), jnp.int32),
            pltpu.VMEM((C,), jnp.int32),
        ],
    )
    def k(nt_hbm, src_hbm, dst_hbm, et_out, nt_v, s_v, d_v, e_v):
        cid = lax.axis_index("c")
        sid = lax.axis_index("s")
        wid = sid * NC + cid
        pltpu.sync_copy(nt_hbm, nt_v)

        def chunk(kk, _):
            base = wid * (K * C) + kk * C
            pltpu.sync_copy(src_hbm.at[pl.ds(base, C)], s_v)
            pltpu.sync_copy(dst_hbm.at[pl.ds(base, C)], d_v)

            def grp(j, _):
                sl = pl.ds(j * LANES, LANES)
                tx = plsc.load_gather(nt_v, [s_v[sl]])
                ty = plsc.load_gather(nt_v, [d_v[sl]])
                e_v[sl] = tx * ty + tx + ty
                return 0

            lax.fori_loop(0, C // LANES, grp, 0)
            pltpu.sync_copy(e_v, et_out.at[pl.ds(base, C)])
            return 0

        lax.fori_loop(0, K, chunk, 0)

    return k


def _edge_sc(VPS, EP, K):
    mesh = plsc.VectorSubcoreMesh(core_axis_name="c", subcore_axis_name="s",
                                  num_cores=NC, num_subcores=NS)
    rows = VPS // NS
    KQ = K // 4

    @functools.partial(
        pl.kernel, mesh=mesh,
        compiler_params=pltpu.CompilerParams(needs_layout_passes=False),
        out_type=jax.ShapeDtypeStruct((NC, VPS, F), jnp.float32),
        scratch_types=[
            pltpu.VMEM((4, C), jnp.int32),
            pltpu.VMEM((4, C), jnp.int32),
            pltpu.VMEM((4, C), jnp.int32),
            pltpu.VMEM((2, C, F), jnp.float32),
            pltpu.VMEM((4, C, F), jnp.float32),
            pltpu.VMEM((2, C, F), jnp.float32),
            pltpu.VMEM_SHARED((VPS, F), jnp.float32),
        ] + [pltpu.SemaphoreType.DMA] * 8,
    )
    def k(nn_hbm, rh_hbm, tnew_hbm, src_hbm, dst_hbm, et_hbm, zeros_hbm, agg_out,
          s4, d4, e4, hs2, tf2, rr2, shared,
          si0, si1, sr0, sr1, sg0, sg1, ss0, ss1):
        cid = lax.axis_index("c")
        sid = lax.axis_index("s")
        wid = sid * NC + cid
        si = [si0, si1]
        sr = [sr0, sr1]
        sg = [sg0, sg1]
        ss = [ss0, ss1]
        ebase = wid * (K * C)

        def issue_idx(kq, j):
            base = ebase + kq * C
            pltpu.async_copy(src_hbm.at[pl.ds(base, C)], s4.at[j], si[j % 2])
            pltpu.async_copy(dst_hbm.at[pl.ds(base, C)], d4.at[j], si[j % 2])
            pltpu.async_copy(et_hbm.at[pl.ds(base, C)], e4.at[j], si[j % 2])

        def wait_idx(kq, j):
            base = ebase + kq * C
            pltpu.make_async_copy(src_hbm.at[pl.ds(base, C)], s4.at[j],
                                  si[j % 2]).wait()
            pltpu.make_async_copy(dst_hbm.at[pl.ds(base, C)], d4.at[j],
                                  si[j % 2]).wait()
            pltpu.make_async_copy(et_hbm.at[pl.ds(base, C)], e4.at[j],
                                  si[j % 2]).wait()

        def issue_rh(kq, j):
            base = ebase + kq * C
            pltpu.async_copy(rh_hbm.at[pl.ds(base, C)], rr2.at[j % 2],
                             sr[j % 2])

        def wait_rh(kq, j):
            base = ebase + kq * C
            pltpu.make_async_copy(rh_hbm.at[pl.ds(base, C)], rr2.at[j % 2],
                                  sr[j % 2]).wait()

        def issue_gather(j):
            pltpu.async_copy(nn_hbm.at[s4.at[j]], hs2.at[j % 2], sg[j % 2])
            pltpu.async_copy(tnew_hbm.at[e4.at[j]], tf2.at[j], sg[j % 2])

        def wait_gather(j):
            pltpu.make_async_copy(nn_hbm.at[s4.at[j]], hs2.at[j % 2],
                                  sg[j % 2]).wait()
            pltpu.make_async_copy(tnew_hbm.at[e4.at[j]], tf2.at[j],
                                  sg[j % 2]).wait()

        def wait_scatter(j):
            pltpu.make_async_copy(tf2.at[j], shared.at[d4.at[j]],
                                  ss[j % 2]).wait()

        pltpu.sync_copy(zeros_hbm.at[pl.ds(sid * rows, rows)],
                        shared.at[pl.ds(sid * rows, rows)])
        plsc.subcore_barrier()

        issue_idx(0, 0)
        issue_idx(1, 1)
        issue_rh(0, 0)
        issue_rh(1, 1)
        wait_idx(0, 0)
        issue_gather(0)

        def outer(kk, _):
            for b in range(4):
                kq = kk * 4 + b

                def _wait_sc():
                    wait_scatter((b + 2) % 4)
                if b < 2:
                    pl.when(kk > 0)(_wait_sc)
                else:
                    _wait_sc()

                def _pf_idx():
                    issue_idx(kq + 2, (b + 2) % 4)
                if b < 2:
                    _pf_idx()
                else:
                    pl.when(kk < KQ - 1)(_pf_idx)

                def _gather_ahead():
                    wait_idx(kq + 1, (b + 1) % 4)
                    issue_gather((b + 1) % 4)
                if b < 3:
                    _gather_ahead()
                else:
                    pl.when(kk < KQ - 1)(_gather_ahead)

                wait_rh(kq, b)
                wait_gather(b)

                @plsc.parallel_loop(0, C, unroll=2)
                def _row(r):
                    for g in range(F // LANES):
                        sl = pl.ds(g * LANES, LANES)
                        tf2[b, r, sl] = (hs2[b % 2, r, sl] * rr2[b % 2, r, sl]
                                         + tf2[b, r, sl])

                pltpu.async_copy(tf2.at[b], shared.at[d4.at[b]], ss[b % 2],
                                 add=True)

                def _pf_rh():
                    issue_rh(kq + 2, (b + 2) % 4)
                if b < 2:
                    _pf_rh()
                else:
                    pl.when(kk < KQ - 1)(_pf_rh)
            return 0

        lax.fori_loop(0, KQ, outer, 0)
        wait_scatter(2)
        wait_scatter(3)
        plsc.subcore_barrier()
        pltpu.sync_copy(shared.at[pl.ds(sid * rows, rows)],
                        agg_out.at[cid, pl.ds(sid * rows, rows)])

    return k


def _pad_rows(x, n):
    return jnp.concatenate(
        [x, jnp.zeros((n - x.shape[0],) + x.shape[1:], x.dtype)], axis=0)


@jax.jit
def _run(node_types, edge_index, edge_dists, node_graph_ids, params):
    V = node_types.shape[0]
    E = edge_index.shape[1]
    VP = ((V + 1 + 1023) // 1024) * 1024
    VPS = ((V + 1 + 127) // 128) * 128
    K = ((-(-E // (NW * C)) + 3) // 4) * 4
    EP = NW * C * K

    nt = node_types.astype(jnp.int32)
    src = _pad_rows(edge_index[0].astype(jnp.int32), EP)
    dst = jnp.concatenate([edge_index[1].astype(jnp.int32),
                           jnp.full((EP - E,), VPS - 1, jnp.int32)])
    dists = _pad_rows(edge_dists[:, 0], EP).reshape(EP, 1)
    ntp = _pad_rows(nt, VP)
    gid = jnp.concatenate([node_graph_ids.astype(jnp.int32),
                           jnp.full((VP - V,), NG + 7, jnp.int32)])

    lps = params['layers']
    c3W = jnp.stack([lp['c3_W'] for lp in lps])
    c3b = jnp.stack([lp['c3_b'] for lp in lps])
    e1W = jnp.stack([lp['el1_W'] for lp in lps])
    e1b = jnp.stack([lp['el1_b'] for lp in lps])
    c1W = jnp.stack([_pad_rows(lp['c1_W'], 8) for lp in lps])
    c1b = jnp.stack([lp['c1_b'] for lp in lps])
    c2W = jnp.stack([lp['c2_W'] for lp in lps])
    c2b = jnp.stack([lp['c2_b'] for lp in lps])
    ee = _pad_rows(params['edge_embed'], TP)
    emb = _pad_rows(params['node_embed'], F)

    RBF_DIM = 5
    centers = np.linspace(0.0, 5.0, RBF_DIM).astype(np.float32)
    inv_gap = float(1.0 / (centers[1] - centers[0]))
    cent = jnp.asarray(np.pad(centers, (0, 8 - RBF_DIM)))

    f32 = jnp.float32
    tnews = pl.pallas_call(
        _tables_body,
        out_shape=[jax.ShapeDtypeStruct((TP, F), f32)] * 3,
    )(ee, c3W, c3b, e1W, e1b)

    BR = 2048
    nblk = EP // BR
    rhs = pl.pallas_call(
        functools.partial(_rh_body, inv_gap),
        grid=(nblk,),
        in_specs=[
            pl.BlockSpec((BR, 1), lambda i: (i, 0)),
            pl.BlockSpec((8,), lambda i: (0,)),
            pl.BlockSpec((3, 8, F), lambda i: (0, 0, 0)),
            pl.BlockSpec((3, F), lambda i: (0, 0)),
            pl.BlockSpec((3, F, F), lambda i: (0, 0, 0)),
            pl.BlockSpec((3, F), lambda i: (0, 0)),
        ],
        out_specs=[pl.BlockSpec((BR, F), lambda i: (i, 0))] * 3,
        out_shape=[jax.ShapeDtypeStruct((EP, F), f32)] * 3,
    )(dists, cent, c1W, c1b, c2W, c2b)

    etype = _etype_sc(VP, EP, K)(ntp, src, dst)

    BN = 2048
    nbn = VP // BN
    h0 = pl.pallas_call(
        _h0_body,
        grid=(nbn,),
        in_specs=[
            pl.BlockSpec((BN, 1), lambda i: (i, 0)),
            pl.BlockSpec((F, F), lambda i: (0, 0)),
        ],
        out_specs=pl.BlockSpec((BN, F), lambda i: (i, 0)),
        out_shape=jax.ShapeDtypeStruct((VP, F), f32),
    )(ntp.reshape(VP, 1), emb)

    zeros_vp = jnp.zeros((VPS, F), f32)
    edge_k = _edge_sc(VPS, EP, K)

    h = h0
    feats = [h0]
    for l in range(3):
        lp = lps[l]
        nn = pl.pallas_call(
            _pre_body,
            grid=(nbn,),
            in_specs=[
                pl.BlockSpec((BN, F), lambda i: (i, 0)),
                pl.BlockSpec((F, F), lambda i: (0, 0)),
                pl.BlockSpec((F,), lambda i: (0,)),
            ],
            out_specs=pl.BlockSpec((BN, F), lambda i: (i, 0)),
            out_shape=jax.ShapeDtypeStruct((VP, F), f32),
        )(h, lp['nl1_W'], lp['nl1_b'])

        agg2 = edge_k(nn, rhs[l], tnews[l], src, dst, etype, zeros_vp)
        agg2 = jnp.concatenate(
            [agg2, jnp.zeros((NC, VP - VPS, F), f32)], axis=1)

        h = pl.pallas_call(
            _post_body,
            grid=(nbn,),
            in_specs=[
                pl.BlockSpec((NC, BN, F), lambda i: (0, i, 0)),
                pl.BlockSpec((BN, F), lambda i: (i, 0)),
                pl.BlockSpec((F, F), lambda i: (0, 0)),
                pl.BlockSpec((F,), lambda i: (0,)),
                pl.BlockSpec((F, F), lambda i: (0, 0)),
                pl.BlockSpec((F,), lambda i: (0,)),
            ],
            out_specs=pl.BlockSpec((BN, F), lambda i: (i, 0)),
            out_shape=jax.ShapeDtypeStruct((VP, F), f32),
        )(agg2, h, lp['nl2_W'], lp['nl2_b'], lp['nl3_W'], lp['nl3_b'])
        feats.append(h)

    W4 = params['ro_in_W'].reshape(4, F, HID)
    W2 = jnp.zeros((F, F), f32)
    W2 = W2.at[:HID, 0].set(params['ro_out_W'][:, 0])
    W2 = W2.at[HID, 0].set(params['ro_out_b'][0])
    BG = 1024
    nbg = VP // BG
    gid3 = gid.reshape(nbg, 1, BG)
    out = pl.pallas_call(
        _readout_body,
        grid=(nbg,),
        in_specs=[pl.BlockSpec((BG, F), lambda i: (i, 0))] * 4 + [
            pl.BlockSpec((1, 1, BG), lambda i: (i, 0, 0)),
            pl.BlockSpec((4, F, HID), lambda i: (0, 0, 0)),
            pl.BlockSpec((HID,), lambda i: (0,)),
            pl.BlockSpec((F, F), lambda i: (0, 0)),
        ],
        out_specs=pl.BlockSpec((NG, F), lambda i: (0, 0)),
        out_shape=jax.ShapeDtypeStruct((NG, F), f32),
        scratch_shapes=[pltpu.VMEM((NG, F), f32)],
    )(feats[0], feats[1], feats[2], feats[3], gid3, W4,
      params['ro_in_b'], W2)
    return out[:, :1]


def kernel(node_types, edge_index, edge_dists, node_graph_ids, params):
    return _run(node_types, edge_index, edge_dists, node_graph_ids, params)
